# baseline (speedup 1.0000x reference)
import functools

import jax
import jax.numpy as jnp
from jax.experimental import pallas as pl

F32 = jnp.float32
_B = 64
_N = 26
_NN = _N * _N
_NF = 96
_C0 = 2 * _NF
_GD = _NF // 2
_NK = 5
_EPS = 1e-5
_PREC = jax.lax.Precision.HIGHEST

_INTERPRET = False


def _dot(a, b):
    return jax.lax.dot_general(a, b, (((a.ndim - 1,), (0,)), ((), ())),
                               precision=_PREC, preferred_element_type=F32)


def _lrelu(x):
    return jnp.where(x >= 0, x, 0.01 * x)


def _accum(sum_ref, h):
    s = jnp.concatenate(
        [jnp.sum(h, axis=0, keepdims=True), jnp.sum(h * h, axis=0, keepdims=True)],
        axis=0)
    @pl.when(pl.program_id(0) == 0)
    def _():
        sum_ref[...] = s

    @pl.when(pl.program_id(0) != 0)
    def _():
        sum_ref[...] += s


def _bn_coeffs(sums, count, g, be):
    mean = sums[0] / count
    var = sums[1] / count - mean * mean
    scale = g * jax.lax.rsqrt(var + _EPS)
    shift = be - mean * scale
    return scale.reshape(1, -1), shift.reshape(1, -1)


def _pass_a_body(x, w, b, h_ref, sum_ref, ep, d):
    diff = jnp.abs(x[:, :, None, :] - x[:, None, :, :])
    x0 = diff.reshape(ep * _NN, d)
    h = _dot(x0, w) + b
    h_ref[...] = h
    _accum(sum_ref, h)


def _pass_a_kernel(x_ref, w_ref, b_ref, h_ref, sum_ref, *, ep, d):
    _pass_a_body(x_ref[...], w_ref[...], b_ref[...], h_ref, sum_ref, ep, d)


def _pass_a_cat_kernel(x_ref, gp_ref, gs_ref, gb_ref, w_ref, b_ref,
                       h_ref, sum_ref, xcat_ref, *, ep, d):
    xn = _lrelu(gp_ref[...] * gs_ref[...] + gb_ref[...])
    x = jnp.concatenate([x_ref[...], xn], axis=-1)
    xcat_ref[...] = x
    _pass_a_body(x, w_ref[...], b_ref[...], h_ref, sum_ref, ep, d)


def _run_pass_a(x, w, b, ep):
    d = x.shape[-1]
    grid = _B // ep
    return pl.pallas_call(
        functools.partial(_pass_a_kernel, ep=ep, d=d),
        grid=(grid,),
        in_specs=[
            pl.BlockSpec((ep, _N, d), lambda g: (g, 0, 0)),
            pl.BlockSpec((d, _C0), lambda g: (0, 0)),
            pl.BlockSpec((1, _C0), lambda g: (0, 0)),
        ],
        out_specs=[
            pl.BlockSpec((ep * _NN, _C0), lambda g: (g, 0)),
            pl.BlockSpec((2, _C0), lambda g: (0, 0)),
        ],
        out_shape=[
            jax.ShapeDtypeStruct((_B * _NN, _C0), F32),
            jax.ShapeDtypeStruct((2, _C0), F32),
        ],
        interpret=_INTERPRET,
    )(x, w, b.reshape(1, -1))


def _run_pass_a_cat(x_old, gpre, gs, gb, w, b, ep):
    d_old = x_old.shape[-1]
    d = d_old + _GD
    grid = _B // ep
    h, sums, xcat = pl.pallas_call(
        functools.partial(_pass_a_cat_kernel, ep=ep, d=d),
        grid=(grid,),
        in_specs=[
            pl.BlockSpec((ep, _N, d_old), lambda g: (g, 0, 0)),
            pl.BlockSpec((ep, _N, _GD), lambda g: (g, 0, 0)),
            pl.BlockSpec((1, 1, _GD), lambda g: (0, 0, 0)),
            pl.BlockSpec((1, 1, _GD), lambda g: (0, 0, 0)),
            pl.BlockSpec((d, _C0), lambda g: (0, 0)),
            pl.BlockSpec((1, _C0), lambda g: (0, 0)),
        ],
        out_specs=[
            pl.BlockSpec((ep * _NN, _C0), lambda g: (g, 0)),
            pl.BlockSpec((2, _C0), lambda g: (0, 0)),
            pl.BlockSpec((ep, _N, d), lambda g: (g, 0, 0)),
        ],
        out_shape=[
            jax.ShapeDtypeStruct((_B * _NN, _C0), F32),
            jax.ShapeDtypeStruct((2, _C0), F32),
            jax.ShapeDtypeStruct((_B, _N, d), F32),
        ],
        interpret=_INTERPRET,
    )(x_old, gpre, gs.reshape(1, 1, -1), gb.reshape(1, 1, -1), w, b.reshape(1, -1))
    return h, sums, xcat


def _pass_p_kernel(h_ref, sc_ref, sh_ref, w_ref, b_ref, out_ref, sum_ref):
    x = _lrelu(h_ref[...] * sc_ref[...] + sh_ref[...])
    h = _dot(x, w_ref[...]) + b_ref[...]
    out_ref[...] = h
    _accum(sum_ref, h)


def _run_pass_p(h_prev, sc, sh, w, b, ep):
    cin = h_prev.shape[-1]
    cout = w.shape[-1]
    grid = _B // ep
    return pl.pallas_call(
        _pass_p_kernel,
        grid=(grid,),
        in_specs=[
            pl.BlockSpec((ep * _NN, cin), lambda g: (g, 0)),
            pl.BlockSpec((1, cin), lambda g: (0, 0)),
            pl.BlockSpec((1, cin), lambda g: (0, 0)),
            pl.BlockSpec((cin, cout), lambda g: (0, 0)),
            pl.BlockSpec((1, cout), lambda g: (0, 0)),
        ],
        out_specs=[
            pl.BlockSpec((ep * _NN, cout), lambda g: (g, 0)),
            pl.BlockSpec((2, cout), lambda g: (0, 0)),
        ],
        out_shape=[
            jax.ShapeDtypeStruct((_B * _NN, cout), F32),
            jax.ShapeDtypeStruct((2, cout), F32),
        ],
        interpret=_INTERPRET,
    )(h_prev, sc, sh, w, b.reshape(1, -1))


def _pass_p3_kernel(h_ref, sc_ref, sh_ref, w_ref, b_ref, out_ref):
    x = _lrelu(h_ref[...] * sc_ref[...] + sh_ref[...])
    out_ref[...] = _dot(x, w_ref[...]) + b_ref[...]


def _run_pass_p3(h_prev, sc, sh, w, b, ep):
    cin = h_prev.shape[-1]
    grid = _B // ep
    return pl.pallas_call(
        _pass_p3_kernel,
        grid=(grid,),
        in_specs=[
            pl.BlockSpec((ep * _NN, cin), lambda g: (g, 0)),
            pl.BlockSpec((1, cin), lambda g: (0, 0)),
            pl.BlockSpec((1, cin), lambda g: (0, 0)),
            pl.BlockSpec((cin, 1), lambda g: (0, 0)),
            pl.BlockSpec((1, 1), lambda g: (0, 0)),
        ],
        out_specs=pl.BlockSpec((ep * _NN, 1), lambda g: (g, 0)),
        out_shape=jax.ShapeDtypeStruct((_B * _NN, 1), F32),
        interpret=_INTERPRET,
    )(h_prev, sc, sh, w, b.reshape(1, 1))


def _pass_g_kernel(lg_ref, x_ref, w_ref, b_ref, out_ref, sum_ref, *, ep, d):
    lg = lg_ref[...]
    row = jax.lax.broadcasted_iota(jnp.int32, (_N, _N), 0)
    col = jax.lax.broadcasted_iota(jnp.int32, (_N, _N), 1)
    eye = (row == col).astype(F32)
    lg = lg - 1e8 * eye[None]
    m = jnp.max(lg, axis=-1, keepdims=True)
    e = jnp.exp(lg - m)
    a = e / jnp.sum(e, axis=-1, keepdims=True)
    x = x_ref[...]
    w = w_ref[...]
    b = b_ref[...]
    for i in range(ep):
        ax = _dot(a[i], x[i])
        cat = jnp.concatenate([x[i], ax], axis=-1)
        h = _dot(cat, w) + b
        out_ref[i] = h
        s = jnp.concatenate([jnp.sum(h, axis=0, keepdims=True),
                             jnp.sum(h * h, axis=0, keepdims=True)], axis=0)
        if i == 0:
            @pl.when(pl.program_id(0) == 0)
            def _():
                sum_ref[...] = s

            @pl.when(pl.program_id(0) != 0)
            def _():
                sum_ref[...] += s
        else:
            sum_ref[...] += s


def _run_pass_g(logits, x, w, b, ep):
    d = x.shape[-1]
    grid = _B // ep
    return pl.pallas_call(
        functools.partial(_pass_g_kernel, ep=ep, d=d),
        grid=(grid,),
        in_specs=[
            pl.BlockSpec((ep, _N, _N), lambda g: (g, 0, 0)),
            pl.BlockSpec((ep, _N, d), lambda g: (g, 0, 0)),
            pl.BlockSpec((2 * d, _GD), lambda g: (0, 0)),
            pl.BlockSpec((1, _GD), lambda g: (0, 0)),
        ],
        out_specs=[
            pl.BlockSpec((ep, _N, _GD), lambda g: (g, 0, 0)),
            pl.BlockSpec((2, _GD), lambda g: (0, 0)),
        ],
        out_shape=[
            jax.ShapeDtypeStruct((_B, _N, _GD), F32),
            jax.ShapeDtypeStruct((2, _GD), F32),
        ],
        interpret=_INTERPRET,
    )(logits, x, w, b.reshape(1, -1))


def _pass_f_kernel(h_ref, sc_ref, sh_ref, w4_ref, b4_ref, x_ref, wg_ref, bg_ref,
                   sig_ref, log_ref):
    x4 = _lrelu(h_ref[...] * sc_ref[...] + sh_ref[...])
    h4 = jnp.sum(x4 * w4_ref[...], axis=-1) + b4_ref[0, 0]
    col = jax.lax.broadcasted_iota(jnp.int32, (_B, _N), 1)
    h4 = h4 - 1e8 * (col == 0).astype(F32)
    m = jnp.max(h4, axis=-1, keepdims=True)
    e = jnp.exp(h4 - m)
    a = e / jnp.sum(e, axis=-1, keepdims=True)
    x = x_ref[...]
    ax = jnp.sum(a[:, :, None] * x, axis=1)
    cat = jnp.concatenate([x[:, 0, :], ax], axis=-1)
    logits = _dot(cat, wg_ref[...]) + bg_ref[...]
    log_ref[...] = logits
    sig_ref[...] = 1.0 / (1.0 + jnp.exp(-logits))


def _run_pass_f(h3_row0, sc, sh, w4, b4, x, wg, bg):
    d = x.shape[-1]
    return pl.pallas_call(
        _pass_f_kernel,
        grid=(1,),
        in_specs=[
            pl.BlockSpec((_B, _N, _NF), lambda g: (0, 0, 0)),
            pl.BlockSpec((1, 1, _NF), lambda g: (0, 0, 0)),
            pl.BlockSpec((1, 1, _NF), lambda g: (0, 0, 0)),
            pl.BlockSpec((1, 1, _NF), lambda g: (0, 0, 0)),
            pl.BlockSpec((1, 1), lambda g: (0, 0)),
            pl.BlockSpec((_B, _N, d), lambda g: (0, 0, 0)),
            pl.BlockSpec((2 * d, _NK), lambda g: (0, 0)),
            pl.BlockSpec((1, _NK), lambda g: (0, 0)),
        ],
        out_specs=[
            pl.BlockSpec((_B, _NK), lambda g: (0, 0)),
            pl.BlockSpec((_B, _NK), lambda g: (0, 0)),
        ],
        out_shape=[
            jax.ShapeDtypeStruct((_B, _NK), F32),
            jax.ShapeDtypeStruct((_B, _NK), F32),
        ],
        interpret=_INTERPRET,
    )(h3_row0, sc.reshape(1, 1, -1), sh.reshape(1, 1, -1), w4.reshape(1, 1, -1),
      b4.reshape(1, 1), x, wg, bg.reshape(1, -1))


_M_PAIR = float(_B * _NN)
_M_NODE = float(_B * _N)
_EP = 4
_EP_G = 16


def _wcompute_mlp(h0, sums0, p):
    sc0, sh0 = _bn_coeffs(sums0, _M_PAIR, p['g0'], p['be0'])
    h1, sums1 = _run_pass_p(h0, sc0, sh0, p['w1'], p['b1'], _EP)
    sc1, sh1 = _bn_coeffs(sums1, _M_PAIR, p['g1'], p['be1'])
    h2, sums2 = _run_pass_p(h1, sc1, sh1, p['w2'], p['b2'], _EP)
    sc2, sh2 = _bn_coeffs(sums2, _M_PAIR, p['g2'], p['be2'])
    h3, sums3 = _run_pass_p(h2, sc2, sh2, p['w3'], p['b3'], _EP)
    sc3, sh3 = _bn_coeffs(sums3, _M_PAIR, p['g3'], p['be3'])
    return h3, sc3, sh3


def kernel(z, zi_s, labels_yi, zero_pad, params):
    labels = jnp.concatenate([zero_pad[None], labels_yi], axis=0)
    feats = jnp.concatenate([z[None], zi_s], axis=0)
    nodes = jnp.concatenate([feats, labels], axis=2)
    x0 = jnp.transpose(nodes, (1, 0, 2))

    p = params['wc0']
    h0, sums0 = _run_pass_a(x0, p['w0'], p['b0'], _EP)
    h3, sc3, sh3 = _wcompute_mlp(h0, sums0, p)
    lg = _run_pass_p3(h3, sc3, sh3, p['w4'], p['b4'], _EP)
    lg = lg.reshape(_B, _N, _N)
    gp = params['gc0']
    gpre0, gsum0 = _run_pass_g(lg, x0, gp['w'], gp['b'], _EP_G)
    gs0, gb0 = _bn_coeffs(gsum0, _M_NODE, gp['g'], gp['be'])

    p = params['wc1']
    h0, sums0, x1 = _run_pass_a_cat(x0, gpre0, gs0, gb0, p['w0'], p['b0'], _EP)
    h3, sc3, sh3 = _wcompute_mlp(h0, sums0, p)
    lg = _run_pass_p3(h3, sc3, sh3, p['w4'], p['b4'], _EP)
    lg = lg.reshape(_B, _N, _N)
    gp = params['gc1']
    gpre1, gsum1 = _run_pass_g(lg, x1, gp['w'], gp['b'], _EP_G)
    gs1, gb1 = _bn_coeffs(gsum1, _M_NODE, gp['g'], gp['be'])

    p = params['wcl']
    h0, sums0, x2 = _run_pass_a_cat(x1, gpre1, gs1, gb1, p['w0'], p['b0'], _EP)
    h3, sc3, sh3 = _wcompute_mlp(h0, sums0, p)
    h3_row0 = h3.reshape(_B, _NN, _NF)[:, :_N, :]
    gp = params['gcl']
    sig, logits = _run_pass_f(h3_row0, sc3, sh3, p['w4'], p['b4'], x2,
                              gp['w'], gp['b'])
    return (sig, logits)


# speedup vs baseline: 1.4380x; 1.4380x over previous
import functools

import jax
import jax.numpy as jnp
from jax.experimental import pallas as pl

F32 = jnp.float32
_B = 64
_N = 26
_NN = _N * _N
_NF = 96
_C0 = 2 * _NF
_GD = _NF // 2
_NK = 5
_EPS = 1e-5
_PREC = jax.lax.Precision.DEFAULT

_INTERPRET = False


def _dot(a, b):
    return jax.lax.dot_general(a, b, (((a.ndim - 1,), (0,)), ((), ())),
                               precision=_PREC, preferred_element_type=F32)


def _lrelu(x):
    return jnp.where(x >= 0, x, 0.01 * x)


def _accum(sum_ref, h):
    s = jnp.concatenate(
        [jnp.sum(h, axis=0, keepdims=True), jnp.sum(h * h, axis=0, keepdims=True)],
        axis=0)
    @pl.when(pl.program_id(0) == 0)
    def _():
        sum_ref[...] = s

    @pl.when(pl.program_id(0) != 0)
    def _():
        sum_ref[...] += s


def _bn_coeffs(sums, count, g, be):
    mean = sums[0] / count
    var = sums[1] / count - mean * mean
    scale = g * jax.lax.rsqrt(var + _EPS)
    shift = be - mean * scale
    return scale.reshape(1, -1), shift.reshape(1, -1)


def _pass_a_body(x, w, b, h_ref, sum_ref, ep, d):
    diff = jnp.abs(x[:, :, None, :] - x[:, None, :, :])
    x0 = diff.reshape(ep * _NN, d)
    h = _dot(x0, w) + b
    h_ref[...] = h
    _accum(sum_ref, h)


def _pass_a_kernel(x_ref, w_ref, b_ref, h_ref, sum_ref, *, ep, d):
    _pass_a_body(x_ref[...], w_ref[...], b_ref[...], h_ref, sum_ref, ep, d)


def _pass_a_cat_kernel(x_ref, gp_ref, gs_ref, gb_ref, w_ref, b_ref,
                       h_ref, sum_ref, xcat_ref, *, ep, d):
    xn = _lrelu(gp_ref[...] * gs_ref[...] + gb_ref[...])
    x = jnp.concatenate([x_ref[...], xn], axis=-1)
    xcat_ref[...] = x
    _pass_a_body(x, w_ref[...], b_ref[...], h_ref, sum_ref, ep, d)


def _run_pass_a(x, w, b, ep):
    d = x.shape[-1]
    grid = _B // ep
    return pl.pallas_call(
        functools.partial(_pass_a_kernel, ep=ep, d=d),
        grid=(grid,),
        in_specs=[
            pl.BlockSpec((ep, _N, d), lambda g: (g, 0, 0)),
            pl.BlockSpec((d, _C0), lambda g: (0, 0)),
            pl.BlockSpec((1, _C0), lambda g: (0, 0)),
        ],
        out_specs=[
            pl.BlockSpec((ep * _NN, _C0), lambda g: (g, 0)),
            pl.BlockSpec((2, _C0), lambda g: (0, 0)),
        ],
        out_shape=[
            jax.ShapeDtypeStruct((_B * _NN, _C0), F32),
            jax.ShapeDtypeStruct((2, _C0), F32),
        ],
        interpret=_INTERPRET,
    )(x, w, b.reshape(1, -1))


def _run_pass_a_cat(x_old, gpre, gs, gb, w, b, ep):
    d_old = x_old.shape[-1]
    d = d_old + _GD
    grid = _B // ep
    h, sums, xcat = pl.pallas_call(
        functools.partial(_pass_a_cat_kernel, ep=ep, d=d),
        grid=(grid,),
        in_specs=[
            pl.BlockSpec((ep, _N, d_old), lambda g: (g, 0, 0)),
            pl.BlockSpec((ep, _N, _GD), lambda g: (g, 0, 0)),
            pl.BlockSpec((1, 1, _GD), lambda g: (0, 0, 0)),
            pl.BlockSpec((1, 1, _GD), lambda g: (0, 0, 0)),
            pl.BlockSpec((d, _C0), lambda g: (0, 0)),
            pl.BlockSpec((1, _C0), lambda g: (0, 0)),
        ],
        out_specs=[
            pl.BlockSpec((ep * _NN, _C0), lambda g: (g, 0)),
            pl.BlockSpec((2, _C0), lambda g: (0, 0)),
            pl.BlockSpec((ep, _N, d), lambda g: (g, 0, 0)),
        ],
        out_shape=[
            jax.ShapeDtypeStruct((_B * _NN, _C0), F32),
            jax.ShapeDtypeStruct((2, _C0), F32),
            jax.ShapeDtypeStruct((_B, _N, d), F32),
        ],
        interpret=_INTERPRET,
    )(x_old, gpre, gs.reshape(1, 1, -1), gb.reshape(1, 1, -1), w, b.reshape(1, -1))
    return h, sums, xcat


def _pass_p_kernel(h_ref, sc_ref, sh_ref, w_ref, b_ref, out_ref, sum_ref):
    x = _lrelu(h_ref[...] * sc_ref[...] + sh_ref[...])
    h = _dot(x, w_ref[...]) + b_ref[...]
    out_ref[...] = h
    _accum(sum_ref, h)


def _run_pass_p(h_prev, sc, sh, w, b, ep):
    cin = h_prev.shape[-1]
    cout = w.shape[-1]
    grid = _B // ep
    return pl.pallas_call(
        _pass_p_kernel,
        grid=(grid,),
        in_specs=[
            pl.BlockSpec((ep * _NN, cin), lambda g: (g, 0)),
            pl.BlockSpec((1, cin), lambda g: (0, 0)),
            pl.BlockSpec((1, cin), lambda g: (0, 0)),
            pl.BlockSpec((cin, cout), lambda g: (0, 0)),
            pl.BlockSpec((1, cout), lambda g: (0, 0)),
        ],
        out_specs=[
            pl.BlockSpec((ep * _NN, cout), lambda g: (g, 0)),
            pl.BlockSpec((2, cout), lambda g: (0, 0)),
        ],
        out_shape=[
            jax.ShapeDtypeStruct((_B * _NN, cout), F32),
            jax.ShapeDtypeStruct((2, cout), F32),
        ],
        interpret=_INTERPRET,
    )(h_prev, sc, sh, w, b.reshape(1, -1))


def _pass_p3_kernel(h_ref, sc_ref, sh_ref, w_ref, b_ref, out_ref):
    x = _lrelu(h_ref[...] * sc_ref[...] + sh_ref[...])
    out_ref[...] = _dot(x, w_ref[...]) + b_ref[...]


def _run_pass_p3(h_prev, sc, sh, w, b, ep):
    cin = h_prev.shape[-1]
    grid = _B // ep
    return pl.pallas_call(
        _pass_p3_kernel,
        grid=(grid,),
        in_specs=[
            pl.BlockSpec((ep * _NN, cin), lambda g: (g, 0)),
            pl.BlockSpec((1, cin), lambda g: (0, 0)),
            pl.BlockSpec((1, cin), lambda g: (0, 0)),
            pl.BlockSpec((cin, 1), lambda g: (0, 0)),
            pl.BlockSpec((1, 1), lambda g: (0, 0)),
        ],
        out_specs=pl.BlockSpec((ep * _NN, 1), lambda g: (g, 0)),
        out_shape=jax.ShapeDtypeStruct((_B * _NN, 1), F32),
        interpret=_INTERPRET,
    )(h_prev, sc, sh, w, b.reshape(1, 1))


def _pass_g_kernel(lg_ref, x_ref, w_ref, b_ref, out_ref, sum_ref, *, ep, d):
    lg = lg_ref[...]
    row = jax.lax.broadcasted_iota(jnp.int32, (_N, _N), 0)
    col = jax.lax.broadcasted_iota(jnp.int32, (_N, _N), 1)
    eye = (row == col).astype(F32)
    lg = lg - 1e8 * eye[None]
    m = jnp.max(lg, axis=-1, keepdims=True)
    e = jnp.exp(lg - m)
    a = e / jnp.sum(e, axis=-1, keepdims=True)
    x = x_ref[...]
    w = w_ref[...]
    b = b_ref[...]
    for i in range(ep):
        ax = _dot(a[i], x[i])
        cat = jnp.concatenate([x[i], ax], axis=-1)
        h = _dot(cat, w) + b
        out_ref[i] = h
        s = jnp.concatenate([jnp.sum(h, axis=0, keepdims=True),
                             jnp.sum(h * h, axis=0, keepdims=True)], axis=0)
        if i == 0:
            @pl.when(pl.program_id(0) == 0)
            def _():
                sum_ref[...] = s

            @pl.when(pl.program_id(0) != 0)
            def _():
                sum_ref[...] += s
        else:
            sum_ref[...] += s


def _run_pass_g(logits, x, w, b, ep):
    d = x.shape[-1]
    grid = _B // ep
    return pl.pallas_call(
        functools.partial(_pass_g_kernel, ep=ep, d=d),
        grid=(grid,),
        in_specs=[
            pl.BlockSpec((ep, _N, _N), lambda g: (g, 0, 0)),
            pl.BlockSpec((ep, _N, d), lambda g: (g, 0, 0)),
            pl.BlockSpec((2 * d, _GD), lambda g: (0, 0)),
            pl.BlockSpec((1, _GD), lambda g: (0, 0)),
        ],
        out_specs=[
            pl.BlockSpec((ep, _N, _GD), lambda g: (g, 0, 0)),
            pl.BlockSpec((2, _GD), lambda g: (0, 0)),
        ],
        out_shape=[
            jax.ShapeDtypeStruct((_B, _N, _GD), F32),
            jax.ShapeDtypeStruct((2, _GD), F32),
        ],
        interpret=_INTERPRET,
    )(logits, x, w, b.reshape(1, -1))


def _pass_f_kernel(h_ref, sc_ref, sh_ref, w4_ref, b4_ref, x_ref, wg_ref, bg_ref,
                   sig_ref, log_ref):
    x4 = _lrelu(h_ref[...] * sc_ref[...] + sh_ref[...])
    h4 = jnp.sum(x4 * w4_ref[...], axis=-1) + b4_ref[0, 0]
    col = jax.lax.broadcasted_iota(jnp.int32, (_B, _N), 1)
    h4 = h4 - 1e8 * (col == 0).astype(F32)
    m = jnp.max(h4, axis=-1, keepdims=True)
    e = jnp.exp(h4 - m)
    a = e / jnp.sum(e, axis=-1, keepdims=True)
    x = x_ref[...]
    ax = jnp.sum(a[:, :, None] * x, axis=1)
    cat = jnp.concatenate([x[:, 0, :], ax], axis=-1)
    logits = _dot(cat, wg_ref[...]) + bg_ref[...]
    log_ref[...] = logits
    sig_ref[...] = 1.0 / (1.0 + jnp.exp(-logits))


def _run_pass_f(h3_row0, sc, sh, w4, b4, x, wg, bg):
    d = x.shape[-1]
    return pl.pallas_call(
        _pass_f_kernel,
        grid=(1,),
        in_specs=[
            pl.BlockSpec((_B, _N, _NF), lambda g: (0, 0, 0)),
            pl.BlockSpec((1, 1, _NF), lambda g: (0, 0, 0)),
            pl.BlockSpec((1, 1, _NF), lambda g: (0, 0, 0)),
            pl.BlockSpec((1, 1, _NF), lambda g: (0, 0, 0)),
            pl.BlockSpec((1, 1), lambda g: (0, 0)),
            pl.BlockSpec((_B, _N, d), lambda g: (0, 0, 0)),
            pl.BlockSpec((2 * d, _NK), lambda g: (0, 0)),
            pl.BlockSpec((1, _NK), lambda g: (0, 0)),
        ],
        out_specs=[
            pl.BlockSpec((_B, _NK), lambda g: (0, 0)),
            pl.BlockSpec((_B, _NK), lambda g: (0, 0)),
        ],
        out_shape=[
            jax.ShapeDtypeStruct((_B, _NK), F32),
            jax.ShapeDtypeStruct((_B, _NK), F32),
        ],
        interpret=_INTERPRET,
    )(h3_row0, sc.reshape(1, 1, -1), sh.reshape(1, 1, -1), w4.reshape(1, 1, -1),
      b4.reshape(1, 1), x, wg, bg.reshape(1, -1))


_M_PAIR = float(_B * _NN)
_M_NODE = float(_B * _N)
_EP = 4
_EP_G = 16


def _wcompute_mlp(h0, sums0, p):
    sc0, sh0 = _bn_coeffs(sums0, _M_PAIR, p['g0'], p['be0'])
    h1, sums1 = _run_pass_p(h0, sc0, sh0, p['w1'], p['b1'], _EP)
    sc1, sh1 = _bn_coeffs(sums1, _M_PAIR, p['g1'], p['be1'])
    h2, sums2 = _run_pass_p(h1, sc1, sh1, p['w2'], p['b2'], _EP)
    sc2, sh2 = _bn_coeffs(sums2, _M_PAIR, p['g2'], p['be2'])
    h3, sums3 = _run_pass_p(h2, sc2, sh2, p['w3'], p['b3'], _EP)
    sc3, sh3 = _bn_coeffs(sums3, _M_PAIR, p['g3'], p['be3'])
    return h3, sc3, sh3


def kernel(z, zi_s, labels_yi, zero_pad, params):
    labels = jnp.concatenate([zero_pad[None], labels_yi], axis=0)
    feats = jnp.concatenate([z[None], zi_s], axis=0)
    nodes = jnp.concatenate([feats, labels], axis=2)
    x0 = jnp.transpose(nodes, (1, 0, 2))

    p = params['wc0']
    h0, sums0 = _run_pass_a(x0, p['w0'], p['b0'], _EP)
    h3, sc3, sh3 = _wcompute_mlp(h0, sums0, p)
    lg = _run_pass_p3(h3, sc3, sh3, p['w4'], p['b4'], _EP)
    lg = lg.reshape(_B, _N, _N)
    gp = params['gc0']
    gpre0, gsum0 = _run_pass_g(lg, x0, gp['w'], gp['b'], _EP_G)
    gs0, gb0 = _bn_coeffs(gsum0, _M_NODE, gp['g'], gp['be'])

    p = params['wc1']
    h0, sums0, x1 = _run_pass_a_cat(x0, gpre0, gs0, gb0, p['w0'], p['b0'], _EP)
    h3, sc3, sh3 = _wcompute_mlp(h0, sums0, p)
    lg = _run_pass_p3(h3, sc3, sh3, p['w4'], p['b4'], _EP)
    lg = lg.reshape(_B, _N, _N)
    gp = params['gc1']
    gpre1, gsum1 = _run_pass_g(lg, x1, gp['w'], gp['b'], _EP_G)
    gs1, gb1 = _bn_coeffs(gsum1, _M_NODE, gp['g'], gp['be'])

    p = params['wcl']
    h0, sums0, x2 = _run_pass_a_cat(x1, gpre1, gs1, gb1, p['w0'], p['b0'], _EP)
    h3, sc3, sh3 = _wcompute_mlp(h0, sums0, p)
    h3_row0 = h3.reshape(_B, _NN, _NF)[:, :_N, :]
    gp = params['gcl']
    sig, logits = _run_pass_f(h3_row0, sc3, sh3, p['w4'], p['b4'], x2,
                              gp['w'], gp['b'])
    return (sig, logits)


# speedup vs baseline: 1.6566x; 1.1520x over previous
import functools

import jax
import jax.numpy as jnp
from jax.experimental import pallas as pl

F32 = jnp.float32
BF16 = jnp.bfloat16
_B = 64
_N = 26
_NN = _N * _N
_NF = 96
_C0 = 2 * _NF
_GD = _NF // 2
_NK = 5
_EPS = 1e-5
_PREC = jax.lax.Precision.DEFAULT

_INTERPRET = False


def _dot(a, b):
    return jax.lax.dot_general(a, b, (((a.ndim - 1,), (0,)), ((), ())),
                               precision=_PREC, preferred_element_type=F32)


def _lrelu(x):
    return jnp.where(x >= 0, x, 0.01 * x)


def _accum(sum_ref, h):
    s = jnp.concatenate(
        [jnp.sum(h, axis=0, keepdims=True), jnp.sum(h * h, axis=0, keepdims=True)],
        axis=0)
    @pl.when(pl.program_id(0) == 0)
    def _():
        sum_ref[...] = s

    @pl.when(pl.program_id(0) != 0)
    def _():
        sum_ref[...] += s


def _bn_coeffs(sums, count, g, be):
    mean = sums[0] / count
    var = sums[1] / count - mean * mean
    scale = g * jax.lax.rsqrt(var + _EPS)
    shift = be - mean * scale
    return scale.reshape(1, -1), shift.reshape(1, -1)


def _pass_a_body(x, w, b, h_ref, sum_ref, ep, d):
    diff = jnp.abs(x[:, :, None, :] - x[:, None, :, :])
    x0 = diff.reshape(ep * _NN, d)
    h = _dot(x0, w) + b
    h_ref[...] = h.astype(h_ref.dtype)
    _accum(sum_ref, h)


def _pass_a_kernel(x_ref, w_ref, b_ref, h_ref, sum_ref, *, ep, d):
    _pass_a_body(x_ref[...], w_ref[...], b_ref[...], h_ref, sum_ref, ep, d)


def _pass_a_cat_kernel(x_ref, gp_ref, gs_ref, gb_ref, w_ref, b_ref,
                       h_ref, sum_ref, xcat_ref, *, ep, d):
    xn = _lrelu(gp_ref[...] * gs_ref[...] + gb_ref[...])
    x = jnp.concatenate([x_ref[...], xn], axis=-1)
    xcat_ref[...] = x
    _pass_a_body(x, w_ref[...], b_ref[...], h_ref, sum_ref, ep, d)


def _run_pass_a(x, w, b, ep):
    d = x.shape[-1]
    grid = _B // ep
    return pl.pallas_call(
        functools.partial(_pass_a_kernel, ep=ep, d=d),
        grid=(grid,),
        in_specs=[
            pl.BlockSpec((ep, _N, d), lambda g: (g, 0, 0)),
            pl.BlockSpec((d, _C0), lambda g: (0, 0)),
            pl.BlockSpec((1, _C0), lambda g: (0, 0)),
        ],
        out_specs=[
            pl.BlockSpec((ep * _NN, _C0), lambda g: (g, 0)),
            pl.BlockSpec((2, _C0), lambda g: (0, 0)),
        ],
        out_shape=[
            jax.ShapeDtypeStruct((_B * _NN, _C0), BF16),
            jax.ShapeDtypeStruct((2, _C0), F32),
        ],
        interpret=_INTERPRET,
    )(x, w, b.reshape(1, -1))


def _run_pass_a_cat(x_old, gpre, gs, gb, w, b, ep):
    d_old = x_old.shape[-1]
    d = d_old + _GD
    grid = _B // ep
    h, sums, xcat = pl.pallas_call(
        functools.partial(_pass_a_cat_kernel, ep=ep, d=d),
        grid=(grid,),
        in_specs=[
            pl.BlockSpec((ep, _N, d_old), lambda g: (g, 0, 0)),
            pl.BlockSpec((ep, _N, _GD), lambda g: (g, 0, 0)),
            pl.BlockSpec((1, 1, _GD), lambda g: (0, 0, 0)),
            pl.BlockSpec((1, 1, _GD), lambda g: (0, 0, 0)),
            pl.BlockSpec((d, _C0), lambda g: (0, 0)),
            pl.BlockSpec((1, _C0), lambda g: (0, 0)),
        ],
        out_specs=[
            pl.BlockSpec((ep * _NN, _C0), lambda g: (g, 0)),
            pl.BlockSpec((2, _C0), lambda g: (0, 0)),
            pl.BlockSpec((ep, _N, d), lambda g: (g, 0, 0)),
        ],
        out_shape=[
            jax.ShapeDtypeStruct((_B * _NN, _C0), BF16),
            jax.ShapeDtypeStruct((2, _C0), F32),
            jax.ShapeDtypeStruct((_B, _N, d), F32),
        ],
        interpret=_INTERPRET,
    )(x_old, gpre, gs.reshape(1, 1, -1), gb.reshape(1, 1, -1), w, b.reshape(1, -1))
    return h, sums, xcat


def _pass_p_kernel(h_ref, sc_ref, sh_ref, w_ref, b_ref, out_ref, sum_ref):
    x = _lrelu(h_ref[...].astype(F32) * sc_ref[...] + sh_ref[...])
    h = _dot(x, w_ref[...]) + b_ref[...]
    out_ref[...] = h.astype(out_ref.dtype)
    _accum(sum_ref, h)


def _run_pass_p(h_prev, sc, sh, w, b, ep):
    cin = h_prev.shape[-1]
    cout = w.shape[-1]
    grid = _B // ep
    return pl.pallas_call(
        _pass_p_kernel,
        grid=(grid,),
        in_specs=[
            pl.BlockSpec((ep * _NN, cin), lambda g: (g, 0)),
            pl.BlockSpec((1, cin), lambda g: (0, 0)),
            pl.BlockSpec((1, cin), lambda g: (0, 0)),
            pl.BlockSpec((cin, cout), lambda g: (0, 0)),
            pl.BlockSpec((1, cout), lambda g: (0, 0)),
        ],
        out_specs=[
            pl.BlockSpec((ep * _NN, cout), lambda g: (g, 0)),
            pl.BlockSpec((2, cout), lambda g: (0, 0)),
        ],
        out_shape=[
            jax.ShapeDtypeStruct((_B * _NN, cout), BF16),
            jax.ShapeDtypeStruct((2, cout), F32),
        ],
        interpret=_INTERPRET,
    )(h_prev, sc, sh, w, b.reshape(1, -1))


def _pass_p3_kernel(h_ref, sc_ref, sh_ref, w_ref, b_ref, out_ref):
    x = _lrelu(h_ref[...].astype(F32) * sc_ref[...] + sh_ref[...])
    out_ref[...] = _dot(x, w_ref[...]) + b_ref[...]


def _run_pass_p3(h_prev, sc, sh, w, b, ep):
    cin = h_prev.shape[-1]
    grid = _B // ep
    return pl.pallas_call(
        _pass_p3_kernel,
        grid=(grid,),
        in_specs=[
            pl.BlockSpec((ep * _NN, cin), lambda g: (g, 0)),
            pl.BlockSpec((1, cin), lambda g: (0, 0)),
            pl.BlockSpec((1, cin), lambda g: (0, 0)),
            pl.BlockSpec((cin, 1), lambda g: (0, 0)),
            pl.BlockSpec((1, 1), lambda g: (0, 0)),
        ],
        out_specs=pl.BlockSpec((ep * _NN, 1), lambda g: (g, 0)),
        out_shape=jax.ShapeDtypeStruct((_B * _NN, 1), F32),
        interpret=_INTERPRET,
    )(h_prev, sc, sh, w, b.reshape(1, 1))


def _pass_g_kernel(lg_ref, x_ref, w_ref, b_ref, out_ref, sum_ref, *, ep, d):
    lg = lg_ref[...]
    row = jax.lax.broadcasted_iota(jnp.int32, (_N, _N), 0)
    col = jax.lax.broadcasted_iota(jnp.int32, (_N, _N), 1)
    eye = (row == col).astype(F32)
    lg = lg - 1e8 * eye[None]
    m = jnp.max(lg, axis=-1, keepdims=True)
    e = jnp.exp(lg - m)
    a = e / jnp.sum(e, axis=-1, keepdims=True)
    x = x_ref[...]
    w = w_ref[...]
    b = b_ref[...]
    for i in range(ep):
        ax = _dot(a[i], x[i])
        cat = jnp.concatenate([x[i], ax], axis=-1)
        h = _dot(cat, w) + b
        out_ref[i] = h
        s = jnp.concatenate([jnp.sum(h, axis=0, keepdims=True),
                             jnp.sum(h * h, axis=0, keepdims=True)], axis=0)
        if i == 0:
            @pl.when(pl.program_id(0) == 0)
            def _():
                sum_ref[...] = s

            @pl.when(pl.program_id(0) != 0)
            def _():
                sum_ref[...] += s
        else:
            sum_ref[...] += s


def _run_pass_g(logits, x, w, b, ep):
    d = x.shape[-1]
    grid = _B // ep
    return pl.pallas_call(
        functools.partial(_pass_g_kernel, ep=ep, d=d),
        grid=(grid,),
        in_specs=[
            pl.BlockSpec((ep, _N, _N), lambda g: (g, 0, 0)),
            pl.BlockSpec((ep, _N, d), lambda g: (g, 0, 0)),
            pl.BlockSpec((2 * d, _GD), lambda g: (0, 0)),
            pl.BlockSpec((1, _GD), lambda g: (0, 0)),
        ],
        out_specs=[
            pl.BlockSpec((ep, _N, _GD), lambda g: (g, 0, 0)),
            pl.BlockSpec((2, _GD), lambda g: (0, 0)),
        ],
        out_shape=[
            jax.ShapeDtypeStruct((_B, _N, _GD), F32),
            jax.ShapeDtypeStruct((2, _GD), F32),
        ],
        interpret=_INTERPRET,
    )(logits, x, w, b.reshape(1, -1))


def _pass_f_kernel(h_ref, sc_ref, sh_ref, w4_ref, b4_ref, x_ref, wg_ref, bg_ref,
                   sig_ref, log_ref):
    x4 = _lrelu(h_ref[...].astype(F32) * sc_ref[...] + sh_ref[...])
    h4 = jnp.sum(x4 * w4_ref[...], axis=-1) + b4_ref[0, 0]
    col = jax.lax.broadcasted_iota(jnp.int32, (_B, _N), 1)
    h4 = h4 - 1e8 * (col == 0).astype(F32)
    m = jnp.max(h4, axis=-1, keepdims=True)
    e = jnp.exp(h4 - m)
    a = e / jnp.sum(e, axis=-1, keepdims=True)
    x = x_ref[...]
    ax = jnp.sum(a[:, :, None] * x, axis=1)
    cat = jnp.concatenate([x[:, 0, :], ax], axis=-1)
    logits = _dot(cat, wg_ref[...]) + bg_ref[...]
    log_ref[...] = logits
    sig_ref[...] = 1.0 / (1.0 + jnp.exp(-logits))


def _run_pass_f(h3_row0, sc, sh, w4, b4, x, wg, bg):
    d = x.shape[-1]
    return pl.pallas_call(
        _pass_f_kernel,
        grid=(1,),
        in_specs=[
            pl.BlockSpec((_B, _N, _NF), lambda g: (0, 0, 0)),
            pl.BlockSpec((1, 1, _NF), lambda g: (0, 0, 0)),
            pl.BlockSpec((1, 1, _NF), lambda g: (0, 0, 0)),
            pl.BlockSpec((1, 1, _NF), lambda g: (0, 0, 0)),
            pl.BlockSpec((1, 1), lambda g: (0, 0)),
            pl.BlockSpec((_B, _N, d), lambda g: (0, 0, 0)),
            pl.BlockSpec((2 * d, _NK), lambda g: (0, 0)),
            pl.BlockSpec((1, _NK), lambda g: (0, 0)),
        ],
        out_specs=[
            pl.BlockSpec((_B, _NK), lambda g: (0, 0)),
            pl.BlockSpec((_B, _NK), lambda g: (0, 0)),
        ],
        out_shape=[
            jax.ShapeDtypeStruct((_B, _NK), F32),
            jax.ShapeDtypeStruct((_B, _NK), F32),
        ],
        interpret=_INTERPRET,
    )(h3_row0, sc.reshape(1, 1, -1), sh.reshape(1, 1, -1), w4.reshape(1, 1, -1),
      b4.reshape(1, 1), x, wg, bg.reshape(1, -1))


_M_PAIR = float(_B * _NN)
_M_NODE = float(_B * _N)
_EP = 4
_EP_G = 16


def _wcompute_mlp(h0, sums0, p):
    sc0, sh0 = _bn_coeffs(sums0, _M_PAIR, p['g0'], p['be0'])
    h1, sums1 = _run_pass_p(h0, sc0, sh0, p['w1'], p['b1'], _EP)
    sc1, sh1 = _bn_coeffs(sums1, _M_PAIR, p['g1'], p['be1'])
    h2, sums2 = _run_pass_p(h1, sc1, sh1, p['w2'], p['b2'], _EP)
    sc2, sh2 = _bn_coeffs(sums2, _M_PAIR, p['g2'], p['be2'])
    h3, sums3 = _run_pass_p(h2, sc2, sh2, p['w3'], p['b3'], _EP)
    sc3, sh3 = _bn_coeffs(sums3, _M_PAIR, p['g3'], p['be3'])
    return h3, sc3, sh3


def kernel(z, zi_s, labels_yi, zero_pad, params):
    labels = jnp.concatenate([zero_pad[None], labels_yi], axis=0)
    feats = jnp.concatenate([z[None], zi_s], axis=0)
    nodes = jnp.concatenate([feats, labels], axis=2)
    x0 = jnp.transpose(nodes, (1, 0, 2))

    p = params['wc0']
    h0, sums0 = _run_pass_a(x0, p['w0'], p['b0'], _EP)
    h3, sc3, sh3 = _wcompute_mlp(h0, sums0, p)
    lg = _run_pass_p3(h3, sc3, sh3, p['w4'], p['b4'], _EP)
    lg = lg.reshape(_B, _N, _N)
    gp = params['gc0']
    gpre0, gsum0 = _run_pass_g(lg, x0, gp['w'], gp['b'], _EP_G)
    gs0, gb0 = _bn_coeffs(gsum0, _M_NODE, gp['g'], gp['be'])

    p = params['wc1']
    h0, sums0, x1 = _run_pass_a_cat(x0, gpre0, gs0, gb0, p['w0'], p['b0'], _EP)
    h3, sc3, sh3 = _wcompute_mlp(h0, sums0, p)
    lg = _run_pass_p3(h3, sc3, sh3, p['w4'], p['b4'], _EP)
    lg = lg.reshape(_B, _N, _N)
    gp = params['gc1']
    gpre1, gsum1 = _run_pass_g(lg, x1, gp['w'], gp['b'], _EP_G)
    gs1, gb1 = _bn_coeffs(gsum1, _M_NODE, gp['g'], gp['be'])

    p = params['wcl']
    h0, sums0, x2 = _run_pass_a_cat(x1, gpre1, gs1, gb1, p['w0'], p['b0'], _EP)
    h3, sc3, sh3 = _wcompute_mlp(h0, sums0, p)
    h3_row0 = h3.reshape(_B, _NN, _NF)[:, :_N, :]
    gp = params['gcl']
    sig, logits = _run_pass_f(h3_row0, sc3, sh3, p['w4'], p['b4'], x2,
                              gp['w'], gp['b'])
    return (sig, logits)
